# Initial kernel scaffold; baseline (speedup 1.0000x reference)
import jax
import jax.numpy as jnp
from jax.experimental import pallas as pl


def kernel(hidden_states, temperature, top_p, embd_weight):
    raise NotImplementedError("write your pallas kernel here")


# speedup vs baseline: 31.0085x; 31.0085x over previous
import functools

import jax
import jax.numpy as jnp
from jax import lax
from jax.experimental import pallas as pl
from jax.experimental.pallas import tpu as pltpu
from jax.experimental.pallas import tpu_sc as plsc

_B = 32
_D = 128
_V = 1_000_000
_VB = 8192
_GRID = (_V + _VB - 1) // _VB
_NEG = -1e30

_CH = 8192
_NF = _V // _CH
_TAIL = _V - _NF * _CH


def _mm_body(h_ref, t_ref, w_ref, lg_ref, m_ref, s_ref):
    i = pl.program_id(0)
    blk = lax.dot_general(h_ref[...], w_ref[...],
                          dimension_numbers=(((1,), (1,)), ((), ())),
                          preferred_element_type=jnp.float32)
    blk = blk / t_ref[...]
    col = i * _VB + lax.broadcasted_iota(jnp.int32, (_B, _VB), 1)
    blk = jnp.where(col < _V, blk, _NEG)
    lg_ref[...] = blk

    @pl.when(i == 0)
    def _init():
        m_ref[...] = jnp.full((_B, 1), _NEG, jnp.float32)
        s_ref[...] = jnp.zeros((_B, 1), jnp.float32)

    bm = jnp.max(blk, axis=1, keepdims=True)
    m_old = m_ref[...]
    m_new = jnp.maximum(m_old, bm)
    s_ref[...] = (s_ref[...] * jnp.exp(m_old - m_new)
                  + jnp.sum(jnp.exp(blk - m_new), axis=1, keepdims=True))
    m_ref[...] = m_new


def _matmul(h, t2, w):
    return pl.pallas_call(
        _mm_body,
        grid=(_GRID,),
        in_specs=[
            pl.BlockSpec((_B, _D), lambda i: (0, 0)),
            pl.BlockSpec((_B, 1), lambda i: (0, 0)),
            pl.BlockSpec((_VB, _D), lambda i: (i, 0)),
        ],
        out_specs=[
            pl.BlockSpec((_B, _VB), lambda i: (0, i)),
            pl.BlockSpec((_B, 1), lambda i: (0, 0)),
            pl.BlockSpec((_B, 1), lambda i: (0, 0)),
        ],
        out_shape=[
            jax.ShapeDtypeStruct((_B, _V), jnp.float32),
            jax.ShapeDtypeStruct((_B, 1), jnp.float32),
            jax.ShapeDtypeStruct((_B, 1), jnp.float32),
        ],
    )(h, t2, w)


@functools.cache
def _make_sc_hist(shift, nb, pshift):
    grp = nb // 128
    mesh = plsc.VectorSubcoreMesh(core_axis_name="c", subcore_axis_name="s")
    lane_iota = lambda: lax.iota(jnp.int32, 16)

    def body(lg_hbm, m_hbm, pref_hbm, out_hbm,
             m_v, pref_v, buf0_v, buf1_v, tail_v, hist_v, outbuf_v, sem0, sem1):
        wid = lax.axis_index("s") * 2 + lax.axis_index("c")
        pltpu.sync_copy(m_hbm, m_v.at[pl.ds(0, _B)])
        pltpu.sync_copy(pref_hbm, pref_v.at[pl.ds(0, _B)])
        m_row = jnp.full((16,), m_v[pl.ds(wid, 16)][0], jnp.float32)
        pref_row = jnp.full((16,), pref_v[pl.ds(wid, 16)][0], jnp.int32)

        zero16 = jnp.zeros((16,), jnp.float32)

        def _zero(j, carry):
            for l in range(16):
                hist_v[l, pl.ds(j * 16, 16)] = zero16
            return carry
        lax.fori_loop(0, nb // 16, _zero, 0)

        def _process(src_v, nvec):
            def vb(kk, carry):
                v = src_v[pl.ds(kk * 16, 16)]
                u = plsc.bitcast(v, jnp.uint32)
                negm = lax.shift_right_arithmetic(plsc.bitcast(v, jnp.int32), 31)
                sk = u ^ (plsc.bitcast(negm, jnp.uint32) | jnp.uint32(0x80000000))
                bucket = ((sk >> jnp.uint32(shift)) & jnp.uint32(nb - 1)).astype(jnp.int32)
                w = jnp.exp(v - m_row)
                if pshift is not None:
                    match = (sk >> jnp.uint32(pshift)).astype(jnp.int32) == pref_row
                    w = jnp.where(match, w, 0.0)
                plsc.addupdate_scatter(hist_v, [lane_iota(), bucket], w)
                return carry
            lax.fori_loop(0, nvec, vb, 0)

        pltpu.async_copy(lg_hbm.at[wid, pl.ds(0, _CH)], buf0_v, sem0)
        pltpu.async_copy(lg_hbm.at[wid, pl.ds(_CH, _CH)], buf1_v, sem1)

        def pair(k, carry):
            c0 = 2 * k
            pltpu.make_async_copy(lg_hbm.at[wid, pl.ds(0, _CH)], buf0_v, sem0).wait()
            _process(buf0_v, _CH // 16)

            @pl.when(c0 + 2 < _NF)
            def _():
                pltpu.async_copy(lg_hbm.at[wid, pl.ds((c0 + 2) * _CH, _CH)],
                                 buf0_v, sem0)

            pltpu.make_async_copy(lg_hbm.at[wid, pl.ds(0, _CH)], buf1_v, sem1).wait()
            _process(buf1_v, _CH // 16)

            @pl.when(c0 + 3 < _NF)
            def _():
                pltpu.async_copy(lg_hbm.at[wid, pl.ds((c0 + 3) * _CH, _CH)],
                                 buf1_v, sem1)
            return carry
        lax.fori_loop(0, _NF // 2, pair, 0)

        pltpu.sync_copy(lg_hbm.at[wid, pl.ds(_NF * _CH, _TAIL)], tail_v)
        _process(tail_v, _TAIL // 16)

        def _red(j, carry):
            acc = hist_v[0, pl.ds(j * 16, 16)]
            for l in range(1, 16):
                acc = acc + hist_v[l, pl.ds(j * 16, 16)]
            outbuf_v[j // 8, pl.ds((j % 8) * 16, 16)] = acc
            return carry
        lax.fori_loop(0, nb // 16, _red, 0)
        pltpu.sync_copy(outbuf_v, out_hbm.at[wid])

    return pl.kernel(
        body,
        out_type=jax.ShapeDtypeStruct((_B, grp, 128), jnp.float32),
        mesh=mesh,
        compiler_params=pltpu.CompilerParams(needs_layout_passes=False),
        scratch_types=[
            pltpu.VMEM((_B + 16,), jnp.float32),
            pltpu.VMEM((_B + 16,), jnp.int32),
            pltpu.VMEM((_CH,), jnp.float32),
            pltpu.VMEM((_CH,), jnp.float32),
            pltpu.VMEM((_TAIL,), jnp.float32),
            pltpu.VMEM((16, nb), jnp.float32),
            pltpu.VMEM((grp, 128), jnp.float32),
            pltpu.SemaphoreType.DMA,
            pltpu.SemaphoreType.DMA,
        ],
    )


def _make_select(nb):
    grp = nb // 128

    def body(hist_ref, resid_ref, beta_ref, rout_ref):
        h3 = hist_ref[...]
        resid = resid_ref[...]
        li = lax.broadcasted_iota(jnp.int32, (128, 128), 0)
        lj = lax.broadcasted_iota(jnp.int32, (128, 128), 1)
        tri = (li > lj).astype(jnp.float32)
        ma_l = lax.dot_general(h3, tri,
                               dimension_numbers=(((2,), (0,)), ((), ())),
                               preferred_element_type=jnp.float32)
        gs = jnp.sum(h3, axis=2)
        gi = lax.broadcasted_iota(jnp.int32, (grp, grp), 0)
        gj = lax.broadcasted_iota(jnp.int32, (grp, grp), 1)
        gtri = (gi > gj).astype(jnp.float32)
        ma_g = lax.dot_general(gs, gtri,
                               dimension_numbers=(((1,), (0,)), ((), ())),
                               preferred_element_type=jnp.float32)
        ma = ma_l + ma_g[:, :, None]

        cond = ma <= resid[:, :, None]
        ig = lax.broadcasted_iota(jnp.int32, (_B, grp, 128), 1)
        il = lax.broadcasted_iota(jnp.int32, (_B, grp, 128), 2)
        idx = ig * 128 + il
        cand = jnp.where(cond, idx, nb)
        beta = jnp.min(jnp.min(cand, axis=2), axis=1)[:, None]
        picked = jnp.sum(jnp.where(idx == beta[:, :, None], ma, 0.0),
                         axis=(1, 2))[:, None]
        beta_ref[...] = beta
        rout_ref[...] = resid - picked

    def run(hist, resid):
        return pl.pallas_call(
            body,
            out_shape=[
                jax.ShapeDtypeStruct((_B, 1), jnp.int32),
                jax.ShapeDtypeStruct((_B, 1), jnp.float32),
            ],
        )(hist, resid)

    return run


_select_2048 = _make_select(2048)
_select_1024 = _make_select(1024)


def _final_body(lg_ref, g_ref, c_ref, out_ref, best_sc, bidx_sc):
    i = pl.program_id(0)
    lg = lg_ref[...]
    gv = g_ref[...]
    cu = c_ref[...]
    u = lax.bitcast_convert_type(lg, jnp.uint32)
    negm = lax.shift_right_arithmetic(lax.bitcast_convert_type(lg, jnp.int32), 31)
    sk = u ^ (lax.bitcast_convert_type(negm, jnp.uint32) | jnp.uint32(0x80000000))
    ski = lax.bitcast_convert_type(sk ^ jnp.uint32(0x80000000), jnp.int32)
    ci = lax.bitcast_convert_type(cu ^ jnp.uint32(0x80000000), jnp.int32)
    col = i * _VB + lax.broadcasted_iota(jnp.int32, (_B, _VB), 1)
    kept = (ski >= ci) & (col < _V)
    val = jnp.where(kept, lg + gv, _NEG)
    bmax = jnp.max(val, axis=1, keepdims=True)
    bi = jnp.min(jnp.where(val == bmax, col, _V), axis=1, keepdims=True)

    @pl.when(i == 0)
    def _init():
        best_sc[...] = jnp.full((_B, 1), _NEG, jnp.float32)
        bidx_sc[...] = jnp.zeros((_B, 1), jnp.int32)

    better = bmax > best_sc[...]
    best_sc[...] = jnp.where(better, bmax, best_sc[...])
    bidx_sc[...] = jnp.where(better, bi, bidx_sc[...])

    @pl.when(i == _GRID - 1)
    def _out():
        out_ref[...] = bidx_sc[...]


def _final(lg, g, c):
    return pl.pallas_call(
        _final_body,
        grid=(_GRID,),
        in_specs=[
            pl.BlockSpec((_B, _VB), lambda i: (0, i)),
            pl.BlockSpec((_B, _VB), lambda i: (0, i)),
            pl.BlockSpec((_B, 1), lambda i: (0, 0)),
        ],
        out_specs=pl.BlockSpec((_B, 1), lambda i: (0, 0)),
        out_shape=jax.ShapeDtypeStruct((_B, 1), jnp.int32),
        scratch_shapes=[
            pltpu.VMEM((_B, 1), jnp.float32),
            pltpu.VMEM((_B, 1), jnp.int32),
        ],
    )(lg, g, c)


def kernel(hidden_states, temperature, top_p, embd_weight):
    t2 = temperature.reshape(_B, 1)
    lg, m, s = _matmul(hidden_states, t2, embd_weight)

    resid0 = top_p.reshape(_B, 1) * s
    m1 = m.reshape(_B)
    zeros_pref = jnp.zeros((_B,), jnp.int32)

    hist1 = _make_sc_hist(21, 2048, None)(lg, m1, zeros_pref)
    beta1, resid1 = _select_2048(hist1, resid0)

    hist2 = _make_sc_hist(10, 2048, 21)(lg, m1, beta1.reshape(_B))
    beta2, resid2 = _select_2048(hist2, resid1)

    pref2 = (beta1 * 2048 + beta2).reshape(_B)
    hist3 = _make_sc_hist(0, 1024, 10)(lg, m1, pref2)
    beta3, _ = _select_1024(hist3, resid2)

    c = ((beta1.astype(jnp.uint32) << 21)
         | (beta2.astype(jnp.uint32) << 10)
         | beta3.astype(jnp.uint32))

    g = jax.random.gumbel(jax.random.key(42), (_B, _V), jnp.float32)
    ids = _final(lg, g, c)
    return ids.reshape(_B).astype(jnp.int32)


# speedup vs baseline: 32.4129x; 1.0453x over previous
import functools

import jax
import jax.numpy as jnp
from jax import lax
from jax.experimental import pallas as pl
from jax.experimental.pallas import tpu as pltpu
from jax.experimental.pallas import tpu_sc as plsc

_B = 32
_D = 128
_V = 1_000_000
_VB = 8192
_GRID = (_V + _VB - 1) // _VB
_NEG = -1e30

_CH = 8192
_NF = _V // _CH
_TAIL = _V - _NF * _CH


def _mm_body(h_ref, t_ref, w_ref, lg_ref, m_ref, s_ref):
    i = pl.program_id(0)
    blk = lax.dot_general(h_ref[...], w_ref[...],
                          dimension_numbers=(((1,), (1,)), ((), ())),
                          preferred_element_type=jnp.float32)
    blk = blk / t_ref[...]
    col = i * _VB + lax.broadcasted_iota(jnp.int32, (_B, _VB), 1)
    blk = jnp.where(col < _V, blk, _NEG)
    lg_ref[...] = blk

    @pl.when(i == 0)
    def _init():
        m_ref[...] = jnp.full((_B, 1), _NEG, jnp.float32)
        s_ref[...] = jnp.zeros((_B, 1), jnp.float32)

    bm = jnp.max(blk, axis=1, keepdims=True)
    m_old = m_ref[...]
    m_new = jnp.maximum(m_old, bm)
    s_ref[...] = (s_ref[...] * jnp.exp(m_old - m_new)
                  + jnp.sum(jnp.exp(blk - m_new), axis=1, keepdims=True))
    m_ref[...] = m_new


def _matmul(h, t2, w):
    return pl.pallas_call(
        _mm_body,
        grid=(_GRID,),
        in_specs=[
            pl.BlockSpec((_B, _D), lambda i: (0, 0)),
            pl.BlockSpec((_B, 1), lambda i: (0, 0)),
            pl.BlockSpec((_VB, _D), lambda i: (i, 0)),
        ],
        out_specs=[
            pl.BlockSpec((_B, _VB), lambda i: (0, i)),
            pl.BlockSpec((_B, 1), lambda i: (0, 0)),
            pl.BlockSpec((_B, 1), lambda i: (0, 0)),
        ],
        out_shape=[
            jax.ShapeDtypeStruct((_B, _V), jnp.float32),
            jax.ShapeDtypeStruct((_B, 1), jnp.float32),
            jax.ShapeDtypeStruct((_B, 1), jnp.float32),
        ],
    )(h, t2, w)


@functools.cache
def _make_sc_hist(shift, nb, pshift):
    grp = nb // 128
    mesh = plsc.VectorSubcoreMesh(core_axis_name="c", subcore_axis_name="s")
    lane_iota = lambda: lax.iota(jnp.int32, 16)

    def body(lg_hbm, m_hbm, pref_hbm, out_hbm,
             m_v, pref_v, buf0_v, buf1_v, tail_v, hist_v, outbuf_v, sem0, sem1):
        wid = lax.axis_index("s") * 2 + lax.axis_index("c")
        pltpu.sync_copy(m_hbm, m_v.at[pl.ds(0, _B)])
        pltpu.sync_copy(pref_hbm, pref_v.at[pl.ds(0, _B)])
        m_row = jnp.full((16,), m_v[pl.ds(wid, 16)][0], jnp.float32)
        pref_row = jnp.full((16,), pref_v[pl.ds(wid, 16)][0], jnp.int32)

        zero16 = jnp.zeros((16,), jnp.float32)

        def _zero(j, carry):
            for l in range(16):
                hist_v[l, pl.ds(j * 16, 16)] = zero16
            return carry
        lax.fori_loop(0, nb // 16, _zero, 0)

        def _process(src_v, nvec, unroll):
            def one(off):
                v = src_v[pl.ds(off, 16)]
                u = plsc.bitcast(v, jnp.uint32)
                negm = lax.shift_right_arithmetic(plsc.bitcast(v, jnp.int32), 31)
                sk = u ^ (plsc.bitcast(negm, jnp.uint32) | jnp.uint32(0x80000000))
                bucket = ((sk >> jnp.uint32(shift)) & jnp.uint32(nb - 1)).astype(jnp.int32)
                w = jnp.exp(v - m_row)
                if pshift is not None:
                    match = (sk >> jnp.uint32(pshift)).astype(jnp.int32) == pref_row
                    w = jnp.where(match, w, 0.0)
                plsc.addupdate_scatter(hist_v, [lane_iota(), bucket], w)

            def vb(kk, carry):
                for j in range(unroll):
                    one(kk * (16 * unroll) + j * 16)
                return carry
            lax.fori_loop(0, nvec // unroll, vb, 0)

        pltpu.async_copy(lg_hbm.at[wid, pl.ds(0, _CH)], buf0_v, sem0)
        pltpu.async_copy(lg_hbm.at[wid, pl.ds(_CH, _CH)], buf1_v, sem1)

        def pair(k, carry):
            c0 = 2 * k
            pltpu.make_async_copy(lg_hbm.at[wid, pl.ds(0, _CH)], buf0_v, sem0).wait()
            _process(buf0_v, _CH // 16, 8)

            @pl.when(c0 + 2 < _NF)
            def _():
                pltpu.async_copy(lg_hbm.at[wid, pl.ds((c0 + 2) * _CH, _CH)],
                                 buf0_v, sem0)

            pltpu.make_async_copy(lg_hbm.at[wid, pl.ds(0, _CH)], buf1_v, sem1).wait()
            _process(buf1_v, _CH // 16, 8)

            @pl.when(c0 + 3 < _NF)
            def _():
                pltpu.async_copy(lg_hbm.at[wid, pl.ds((c0 + 3) * _CH, _CH)],
                                 buf1_v, sem1)
            return carry
        lax.fori_loop(0, _NF // 2, pair, 0)

        pltpu.sync_copy(lg_hbm.at[wid, pl.ds(_NF * _CH, _TAIL)], tail_v)
        _process(tail_v, _TAIL // 16, 4)

        def _red(j, carry):
            acc = hist_v[0, pl.ds(j * 16, 16)]
            for l in range(1, 16):
                acc = acc + hist_v[l, pl.ds(j * 16, 16)]
            outbuf_v[j // 8, pl.ds((j % 8) * 16, 16)] = acc
            return carry
        lax.fori_loop(0, nb // 16, _red, 0)
        pltpu.sync_copy(outbuf_v, out_hbm.at[wid])

    return pl.kernel(
        body,
        out_type=jax.ShapeDtypeStruct((_B, grp, 128), jnp.float32),
        mesh=mesh,
        compiler_params=pltpu.CompilerParams(needs_layout_passes=False),
        scratch_types=[
            pltpu.VMEM((_B + 16,), jnp.float32),
            pltpu.VMEM((_B + 16,), jnp.int32),
            pltpu.VMEM((_CH,), jnp.float32),
            pltpu.VMEM((_CH,), jnp.float32),
            pltpu.VMEM((_TAIL,), jnp.float32),
            pltpu.VMEM((16, nb), jnp.float32),
            pltpu.VMEM((grp, 128), jnp.float32),
            pltpu.SemaphoreType.DMA,
            pltpu.SemaphoreType.DMA,
        ],
    )


def _make_select(nb):
    grp = nb // 128

    def body(hist_ref, resid_ref, beta_ref, rout_ref):
        h3 = hist_ref[...]
        resid = resid_ref[...]
        li = lax.broadcasted_iota(jnp.int32, (128, 128), 0)
        lj = lax.broadcasted_iota(jnp.int32, (128, 128), 1)
        tri = (li > lj).astype(jnp.float32)
        ma_l = lax.dot_general(h3, tri,
                               dimension_numbers=(((2,), (0,)), ((), ())),
                               preferred_element_type=jnp.float32)
        gs = jnp.sum(h3, axis=2)
        gi = lax.broadcasted_iota(jnp.int32, (grp, grp), 0)
        gj = lax.broadcasted_iota(jnp.int32, (grp, grp), 1)
        gtri = (gi > gj).astype(jnp.float32)
        ma_g = lax.dot_general(gs, gtri,
                               dimension_numbers=(((1,), (0,)), ((), ())),
                               preferred_element_type=jnp.float32)
        ma = ma_l + ma_g[:, :, None]

        cond = ma <= resid[:, :, None]
        ig = lax.broadcasted_iota(jnp.int32, (_B, grp, 128), 1)
        il = lax.broadcasted_iota(jnp.int32, (_B, grp, 128), 2)
        idx = ig * 128 + il
        cand = jnp.where(cond, idx, nb)
        beta = jnp.min(jnp.min(cand, axis=2), axis=1)[:, None]
        picked = jnp.sum(jnp.where(idx == beta[:, :, None], ma, 0.0),
                         axis=(1, 2))[:, None]
        beta_ref[...] = beta
        rout_ref[...] = resid - picked

    def run(hist, resid):
        return pl.pallas_call(
            body,
            out_shape=[
                jax.ShapeDtypeStruct((_B, 1), jnp.int32),
                jax.ShapeDtypeStruct((_B, 1), jnp.float32),
            ],
        )(hist, resid)

    return run


_select_2048 = _make_select(2048)
_select_1024 = _make_select(1024)


def _final_body(lg_ref, g_ref, c_ref, out_ref, best_sc, bidx_sc):
    i = pl.program_id(0)
    lg = lg_ref[...]
    gv = g_ref[...]
    cu = c_ref[...]
    u = lax.bitcast_convert_type(lg, jnp.uint32)
    negm = lax.shift_right_arithmetic(lax.bitcast_convert_type(lg, jnp.int32), 31)
    sk = u ^ (lax.bitcast_convert_type(negm, jnp.uint32) | jnp.uint32(0x80000000))
    ski = lax.bitcast_convert_type(sk ^ jnp.uint32(0x80000000), jnp.int32)
    ci = lax.bitcast_convert_type(cu ^ jnp.uint32(0x80000000), jnp.int32)
    col = i * _VB + lax.broadcasted_iota(jnp.int32, (_B, _VB), 1)
    kept = (ski >= ci) & (col < _V)
    val = jnp.where(kept, lg + gv, _NEG)
    bmax = jnp.max(val, axis=1, keepdims=True)
    bi = jnp.min(jnp.where(val == bmax, col, _V), axis=1, keepdims=True)

    @pl.when(i == 0)
    def _init():
        best_sc[...] = jnp.full((_B, 1), _NEG, jnp.float32)
        bidx_sc[...] = jnp.zeros((_B, 1), jnp.int32)

    better = bmax > best_sc[...]
    best_sc[...] = jnp.where(better, bmax, best_sc[...])
    bidx_sc[...] = jnp.where(better, bi, bidx_sc[...])

    @pl.when(i == _GRID - 1)
    def _out():
        out_ref[...] = bidx_sc[...]


def _final(lg, g, c):
    return pl.pallas_call(
        _final_body,
        grid=(_GRID,),
        in_specs=[
            pl.BlockSpec((_B, _VB), lambda i: (0, i)),
            pl.BlockSpec((_B, _VB), lambda i: (0, i)),
            pl.BlockSpec((_B, 1), lambda i: (0, 0)),
        ],
        out_specs=pl.BlockSpec((_B, 1), lambda i: (0, 0)),
        out_shape=jax.ShapeDtypeStruct((_B, 1), jnp.int32),
        scratch_shapes=[
            pltpu.VMEM((_B, 1), jnp.float32),
            pltpu.VMEM((_B, 1), jnp.int32),
        ],
    )(lg, g, c)


def kernel(hidden_states, temperature, top_p, embd_weight):
    t2 = temperature.reshape(_B, 1)
    lg, m, s = _matmul(hidden_states, t2, embd_weight)

    resid0 = top_p.reshape(_B, 1) * s
    m1 = m.reshape(_B)
    zeros_pref = jnp.zeros((_B,), jnp.int32)

    hist1 = _make_sc_hist(21, 2048, None)(lg, m1, zeros_pref)
    beta1, resid1 = _select_2048(hist1, resid0)

    hist2 = _make_sc_hist(10, 2048, 21)(lg, m1, beta1.reshape(_B))
    beta2, resid2 = _select_2048(hist2, resid1)

    pref2 = (beta1 * 2048 + beta2).reshape(_B)
    hist3 = _make_sc_hist(0, 1024, 10)(lg, m1, pref2)
    beta3, _ = _select_1024(hist3, resid2)

    c = ((beta1.astype(jnp.uint32) << 21)
         | (beta2.astype(jnp.uint32) << 10)
         | beta3.astype(jnp.uint32))

    g = jax.random.gumbel(jax.random.key(42), (_B, _V), jnp.float32)
    ids = _final(lg, g, c)
    return ids.reshape(_B).astype(jnp.int32)
